# baseline (speedup 1.0000x reference)
import functools

import jax
import jax.numpy as jnp
from jax import lax
from jax.experimental import pallas as pl
from jax.experimental.pallas import tpu as pltpu
from jax.experimental.pallas import tpu_sc as plsc

WIDTH = 32
NSAMPLE = 32
ENC = [32, 64, 128, 256, 512]
DEC = [256, 128, 64, 32]
B, N = 4, 4096


def _stem_body(fin_ref, w_ref, out_ref):
    out_ref[0] = jax.nn.relu(
        jnp.dot(fin_ref[0], w_ref[:], preferred_element_type=jnp.float32))


def _stem(fin, W):
    return pl.pallas_call(
        _stem_body,
        grid=(B,),
        in_specs=[pl.BlockSpec((1, N, 4), lambda b: (b, 0, 0)),
                  pl.BlockSpec((4, WIDTH), lambda b: (0, 0))],
        out_specs=pl.BlockSpec((1, N, WIDTH), lambda b: (b, 0, 0)),
        out_shape=jax.ShapeDtypeStruct((B, N, WIDTH), jnp.float32),
    )(fin, W)


def _sqc_body(p_ref, pc_ref, f_ref, wg_ref, s_ref, qc_ref):
    w_p = wg_ref[:3, :]
    w_f = wg_ref[3:, :]
    s_ref[0] = (jnp.dot(p_ref[0], w_p, preferred_element_type=jnp.float32)
                + jnp.dot(f_ref[0], w_f, preferred_element_type=jnp.float32))
    qc_ref[0] = jnp.dot(pc_ref[0], w_p, preferred_element_type=jnp.float32)


def _sqc(p, pc, f, Wg):
    _, M, _ = p.shape
    Mc = pc.shape[1]
    cin = f.shape[2]
    cout = Wg.shape[1]
    return pl.pallas_call(
        _sqc_body,
        grid=(B,),
        in_specs=[pl.BlockSpec((1, M, 3), lambda b: (b, 0, 0)),
                  pl.BlockSpec((1, Mc, 3), lambda b: (b, 0, 0)),
                  pl.BlockSpec((1, M, cin), lambda b: (b, 0, 0)),
                  pl.BlockSpec((3 + cin, cout), lambda b: (0, 0))],
        out_specs=(pl.BlockSpec((1, M, cout), lambda b: (b, 0, 0)),
                   pl.BlockSpec((1, Mc, cout), lambda b: (b, 0, 0))),
        out_shape=(jax.ShapeDtypeStruct((B, M, cout), jnp.float32),
                   jax.ShapeDtypeStruct((B, Mc, cout), jnp.float32)),
    )(p, pc, f, Wg)


def _knn_body(pc_ref, pT_ref, idx_ref, e_scr, *, M, Mct, k):
    pc = pc_ref[0]
    pT = pT_ref[0]
    d0 = pc[:, 0:1] - pT[0:1, :]
    d1 = pc[:, 1:2] - pT[1:2, :]
    d2c = pc[:, 2:3] - pT[2:3, :]
    e0 = d0 * d0 + d1 * d1 + d2c * d2c
    e_scr[:] = e0
    iota = lax.broadcasted_iota(jnp.int32, (Mct, M), 1)
    lanek = lax.broadcasted_iota(jnp.int32, (Mct, k), 1)
    bM = pl.program_id(0) * M

    m0 = jnp.min(e0, axis=1, keepdims=True)
    am0 = jnp.min(jnp.where(e0 == m0, iota, M), axis=1)

    def it(j, carry):
        am, idxm = carry
        idxm = jnp.where(lanek == j - 1, am[:, None] + bM, idxm)
        e = jnp.where(iota == am[:, None], jnp.inf, e_scr[:])
        e_scr[:] = e
        m = jnp.min(e, axis=1, keepdims=True)
        am = jnp.min(jnp.where(e == m, iota, M), axis=1)
        return am, idxm

    am_last, idxm = lax.fori_loop(
        1, k, it, (am0, jnp.zeros((Mct, k), jnp.int32)))
    idx_ref[0] = jnp.where(lanek == k - 1, am_last[:, None] + bM, idxm)


def _knn_topk(pc, pT, k, Mct):
    _, Mc, _ = pc.shape
    M = pT.shape[2]
    T = Mc // Mct
    body = functools.partial(_knn_body, M=M, Mct=Mct, k=k)
    return pl.pallas_call(
        body,
        grid=(B, T),
        in_specs=[pl.BlockSpec((1, Mct, 3), lambda b, t: (b, t, 0)),
                  pl.BlockSpec((1, 3, M), lambda b, t: (b, 0, 0))],
        out_specs=pl.BlockSpec((1, Mct, k), lambda b, t: (b, t, 0)),
        out_shape=jax.ShapeDtypeStruct((B, Mc, k), jnp.int32),
        scratch_shapes=[pltpu.VMEM((Mct, M), jnp.float32)],
    )(pc, pT)


def _make_sc_gathermax(R, C, ncb, nblk):
    mesh = plsc.VectorSubcoreMesh(core_axis_name="c", subcore_axis_name="s")

    @functools.partial(
        pl.kernel, mesh=mesh,
        out_type=jax.ShapeDtypeStruct((R, C), jnp.float32),
        scratch_types=[pltpu.VMEM((ncb * 32,), jnp.int32),
                       pltpu.VMEM((ncb * 32, C), jnp.float32),
                       pltpu.VMEM((ncb, C), jnp.float32),
                       pltpu.SemaphoreType.DMA],
    )
    def k(s_hbm, idx_hbm, out_hbm, idx_v, rows_v, out_v, sem):
        wid = lax.axis_index("s") * 2 + lax.axis_index("c")

        def blk_body(blk, carry0):
            base = (wid * nblk + blk) * ncb
            pltpu.sync_copy(idx_hbm.at[pl.ds(base * 32, ncb * 32)], idx_v)
            pltpu.async_copy(s_hbm.at[idx_v], rows_v, sem).wait()

            def c_body(c, carry1):
                def q_body(q, carry2):
                    sl = pl.ds(q * 16, 16)
                    acc = rows_v[c * 32, sl]
                    for r in range(1, 32):
                        acc = jnp.maximum(acc, rows_v[c * 32 + r, sl])
                    out_v[c, sl] = acc
                    return carry2

                lax.fori_loop(0, C // 16, q_body, 0)
                return carry1

            lax.fori_loop(0, ncb, c_body, 0)
            pltpu.sync_copy(out_v, out_hbm.at[pl.ds(base, ncb)])
            return carry0

        lax.fori_loop(0, nblk, blk_body, 0)

    return k


def _res_body(m_ref, qc_ref, w1_ref, w2_ref, out_ref):
    fnew = jax.nn.relu(m_ref[0] - qc_ref[0])
    t = jax.nn.relu(jnp.dot(fnew, w1_ref[:], preferred_element_type=jnp.float32))
    out_ref[0] = jax.nn.relu(
        fnew + jnp.dot(t, w2_ref[:], preferred_element_type=jnp.float32))


def _residual(m, qc, W1, W2):
    _, Mc, cout = m.shape
    return pl.pallas_call(
        _res_body,
        grid=(B,),
        in_specs=[pl.BlockSpec((1, Mc, cout), lambda b: (b, 0, 0)),
                  pl.BlockSpec((1, Mc, cout), lambda b: (b, 0, 0)),
                  pl.BlockSpec((cout, 4 * cout), lambda b: (0, 0)),
                  pl.BlockSpec((4 * cout, cout), lambda b: (0, 0))],
        out_specs=pl.BlockSpec((1, Mc, cout), lambda b: (b, 0, 0)),
        out_shape=jax.ShapeDtypeStruct((B, Mc, cout), jnp.float32),
    )(m, qc, W1, W2)


def _dec_body(pf_ref, pcoT_ref, ffc_ref, fsk_ref, wd_ref, out_ref, *, Nc, Nft, Cc):
    pf = pf_ref[0]
    pcoT = pcoT_ref[0]
    d0 = pf[:, 0:1] - pcoT[0:1, :]
    d1 = pf[:, 1:2] - pcoT[1:2, :]
    d2c = pf[:, 2:3] - pcoT[2:3, :]
    e = d0 * d0 + d1 * d1 + d2c * d2c
    iota = lax.broadcasted_iota(jnp.int32, (Nft, Nc), 1)
    WS = jnp.zeros((Nft, Nc), jnp.float32)
    wsum = jnp.zeros((Nft, 1), jnp.float32)
    for _ in range(3):
        m = jnp.min(e, axis=1, keepdims=True)
        am = jnp.min(jnp.where(e == m, iota, Nc), axis=1)
        oh = iota == am[:, None]
        w = 1.0 / (m + 1e-8)
        WS = WS + jnp.where(oh, w, 0.0)
        wsum = wsum + w
        e = jnp.where(oh, jnp.inf, e)
    WS = WS / wsum
    g = jnp.dot(ffc_ref[0], wd_ref[:Cc, :], preferred_element_type=jnp.float32)
    out_ref[0] = jax.nn.relu(
        jnp.dot(WS, g, preferred_element_type=jnp.float32)
        + jnp.dot(fsk_ref[0], wd_ref[Cc:, :], preferred_element_type=jnp.float32))


def _decoder(pf, pcoT, ffc, fsk, Wd, Nft):
    _, Nf, _ = pf.shape
    Nc = pcoT.shape[2]
    Cc = ffc.shape[2]
    Cs = fsk.shape[2]
    Cd = Wd.shape[1]
    T = Nf // Nft
    body = functools.partial(_dec_body, Nc=Nc, Nft=Nft, Cc=Cc)
    return pl.pallas_call(
        body,
        grid=(B, T),
        in_specs=[pl.BlockSpec((1, Nft, 3), lambda b, t: (b, t, 0)),
                  pl.BlockSpec((1, 3, Nc), lambda b, t: (b, 0, 0)),
                  pl.BlockSpec((1, Nc, Cc), lambda b, t: (b, 0, 0)),
                  pl.BlockSpec((1, Nft, Cs), lambda b, t: (b, t, 0)),
                  pl.BlockSpec((Cc + Cs, Cd), lambda b, t: (0, 0))],
        out_specs=pl.BlockSpec((1, Nft, Cd), lambda b, t: (b, t, 0)),
        out_shape=jax.ShapeDtypeStruct((B, Nf, Cd), jnp.float32),
    )(pf, pcoT, ffc, fsk, Wd)


def _head_body(f_ref, cf_ref, w12_ref, wc_ref, out_ref):
    f = f_ref[0]
    f12 = jnp.dot(f, w12_ref[:], preferred_element_type=jnp.float32)
    out_ref[0, :N, :] = f12[:, :64]
    out_ref[0, N:2 * N, :] = f12[:, 64:]
    out_ref[0, 2 * N:, :] = jnp.dot(cf_ref[0], wc_ref[:],
                                    preferred_element_type=jnp.float32)


def _heads(f, cf, W12, Wc):
    Nt = 2 * N + 16
    return pl.pallas_call(
        _head_body,
        grid=(B,),
        in_specs=[pl.BlockSpec((1, N, 32), lambda b: (b, 0, 0)),
                  pl.BlockSpec((1, 16, 512), lambda b: (b, 0, 0)),
                  pl.BlockSpec((32, 128), lambda b: (0, 0)),
                  pl.BlockSpec((512, 64), lambda b: (0, 0))],
        out_specs=pl.BlockSpec((1, Nt, 64), lambda b: (b, 0, 0)),
        out_shape=jax.ShapeDtypeStruct((B, Nt, 64), jnp.float32),
    )(f, cf, W12, Wc)


_KNN_TILE = [256, 256, 64, 16]
_SC_NCB = [8, 8, 4, 2]
_SC_NBLK = [16, 4, 2, 1]


def kernel(x, W_stem, Wg0, Wg1, Wg2, Wg3, Wr1_0, Wr1_1, Wr1_2, Wr1_3,
           Wr2_0, Wr2_1, Wr2_2, Wr2_3, Wd0, Wd1, Wd2, Wd3, W_f1, W_f2, W_c):
    Wg = [Wg0, Wg1, Wg2, Wg3]
    Wr1 = [Wr1_0, Wr1_1, Wr1_2, Wr1_3]
    Wr2 = [Wr2_0, Wr2_1, Wr2_2, Wr2_3]
    Wd = [Wd0, Wd1, Wd2, Wd3]

    fin = jnp.concatenate([x, x[:, :, 2:]], axis=-1)
    f = _stem(fin, W_stem)
    p = x
    ps = [x]
    fs = [f]
    for i in range(4):
        M = p.shape[1]
        Mc = M // 4
        cout = ENC[i + 1]
        pc = p[:, ::4]
        pT = jnp.swapaxes(p, 1, 2)
        s, qc = _sqc(p, pc, f, Wg[i])
        idx = _knn_topk(pc, pT, NSAMPLE, _KNN_TILE[i])
        Cg = max(cout, 128)
        s_flat = s.reshape(B * M, cout)
        if Cg != cout:
            s_flat = jnp.pad(s_flat, ((0, 0), (0, Cg - cout)))
        gm = _make_sc_gathermax(B * Mc, Cg, _SC_NCB[i], _SC_NBLK[i])
        m = gm(s_flat, idx.reshape(B * Mc * NSAMPLE))
        m = m[:, :cout].reshape(B, Mc, cout)
        f = _residual(m, qc, Wr1[i], Wr2[i])
        p = pc
        ps.append(p)
        fs.append(f)

    c_feats = fs[-1]
    fp, ff = ps[-1], fs[-1]
    for j in range(4):
        p_s, f_s = ps[3 - j], fs[3 - j]
        Nf = p_s.shape[1]
        Nft = min(Nf, 1024)
        pcoT = jnp.swapaxes(fp, 1, 2)
        ff = _decoder(p_s, pcoT, ff, f_s, Wd[j], Nft)
        fp = p_s

    return _heads(ff, c_feats, jnp.concatenate([W_f1, W_f2], axis=1), W_c)


# speedup vs baseline: 1.0640x; 1.0640x over previous
import functools

import jax
import jax.numpy as jnp
from jax import lax
from jax.experimental import pallas as pl
from jax.experimental.pallas import tpu as pltpu
from jax.experimental.pallas import tpu_sc as plsc

WIDTH = 32
NSAMPLE = 32
ENC = [32, 64, 128, 256, 512]
DEC = [256, 128, 64, 32]
B, N = 4, 4096


def _stem_body(fin_ref, w_ref, out_ref):
    out_ref[0] = jax.nn.relu(
        jnp.dot(fin_ref[0], w_ref[:], preferred_element_type=jnp.float32))


def _stem(fin, W):
    return pl.pallas_call(
        _stem_body,
        grid=(B,),
        in_specs=[pl.BlockSpec((1, N, 4), lambda b: (b, 0, 0)),
                  pl.BlockSpec((4, WIDTH), lambda b: (0, 0))],
        out_specs=pl.BlockSpec((1, N, WIDTH), lambda b: (b, 0, 0)),
        out_shape=jax.ShapeDtypeStruct((B, N, WIDTH), jnp.float32),
    )(fin, W)


def _sqc_body(p_ref, pc_ref, f_ref, wg_ref, s_ref, qc_ref):
    w_p = wg_ref[:3, :]
    w_f = wg_ref[3:, :]
    s_ref[0] = (jnp.dot(p_ref[0], w_p, preferred_element_type=jnp.float32)
                + jnp.dot(f_ref[0], w_f, preferred_element_type=jnp.float32))
    qc_ref[0] = jnp.dot(pc_ref[0], w_p, preferred_element_type=jnp.float32)


def _sqc(p, pc, f, Wg):
    _, M, _ = p.shape
    Mc = pc.shape[1]
    cin = f.shape[2]
    cout = Wg.shape[1]
    return pl.pallas_call(
        _sqc_body,
        grid=(B,),
        in_specs=[pl.BlockSpec((1, M, 3), lambda b: (b, 0, 0)),
                  pl.BlockSpec((1, Mc, 3), lambda b: (b, 0, 0)),
                  pl.BlockSpec((1, M, cin), lambda b: (b, 0, 0)),
                  pl.BlockSpec((3 + cin, cout), lambda b: (0, 0))],
        out_specs=(pl.BlockSpec((1, M, cout), lambda b: (b, 0, 0)),
                   pl.BlockSpec((1, Mc, cout), lambda b: (b, 0, 0))),
        out_shape=(jax.ShapeDtypeStruct((B, M, cout), jnp.float32),
                   jax.ShapeDtypeStruct((B, Mc, cout), jnp.float32)),
    )(p, pc, f, Wg)


def _knn_body(pc_ref, pT_ref, idx_ref, e_scr, *, M, Mct, k):
    pc = pc_ref[0]
    pT = pT_ref[0]
    d0 = pc[:, 0:1] - pT[0:1, :]
    d1 = pc[:, 1:2] - pT[1:2, :]
    d2c = pc[:, 2:3] - pT[2:3, :]
    e_scr[:] = d0 * d0 + d1 * d1 + d2c * d2c
    iota = lax.broadcasted_iota(jnp.int32, (Mct, M), 1)
    lanek = lax.broadcasted_iota(jnp.int32, (Mct, k), 1)
    bM = pl.program_id(0) * M

    def it(j, idxm):
        e = e_scr[:]
        m = jnp.min(e, axis=1, keepdims=True)
        am = jnp.min(jnp.where(e == m, iota, M), axis=1)
        e_scr[:] = jnp.where(iota == am[:, None], jnp.inf, e)
        return jnp.where(lanek == j, am[:, None] + bM, idxm)

    idx_ref[0] = lax.fori_loop(0, k, it, jnp.zeros((Mct, k), jnp.int32))


def _knn_topk(pc, pT, k, Mct):
    _, Mc, _ = pc.shape
    M = pT.shape[2]
    T = Mc // Mct
    body = functools.partial(_knn_body, M=M, Mct=Mct, k=k)
    return pl.pallas_call(
        body,
        grid=(B, T),
        in_specs=[pl.BlockSpec((1, Mct, 3), lambda b, t: (b, t, 0)),
                  pl.BlockSpec((1, 3, M), lambda b, t: (b, 0, 0))],
        out_specs=pl.BlockSpec((1, Mct, k), lambda b, t: (b, t, 0)),
        out_shape=jax.ShapeDtypeStruct((B, Mc, k), jnp.int32),
        scratch_shapes=[pltpu.VMEM((Mct, M), jnp.float32)],
    )(pc, pT)


def _make_sc_gathermax(R, C, ncb, nblk):
    mesh = plsc.VectorSubcoreMesh(core_axis_name="c", subcore_axis_name="s")

    nw = ncb * 32 * nblk

    @functools.partial(
        pl.kernel, mesh=mesh,
        out_type=jax.ShapeDtypeStruct((R, C), jnp.float32),
        scratch_types=[pltpu.VMEM((nw,), jnp.int32),
                       pltpu.VMEM((ncb * 32, C), jnp.float32),
                       pltpu.VMEM((ncb * 32, C), jnp.float32),
                       pltpu.VMEM((ncb, C), jnp.float32),
                       pltpu.SemaphoreType.DMA,
                       pltpu.SemaphoreType.DMA],
    )
    def k(s_hbm, idx_hbm, out_hbm, idx_v, rows0, rows1, out_v, sem0, sem1):
        wid = lax.axis_index("s") * 2 + lax.axis_index("c")
        pltpu.sync_copy(idx_hbm.at[pl.ds(wid * nw, nw)], idx_v)
        rows = [rows0, rows1]
        sems = [sem0, sem1]
        cps = [None, None]
        cps[0] = pltpu.async_copy(
            s_hbm.at[idx_v.at[pl.ds(0, ncb * 32)]], rows0, sem0)
        for blk in range(nblk):
            if blk + 1 < nblk:
                b1 = (blk + 1) % 2
                cps[b1] = pltpu.async_copy(
                    s_hbm.at[idx_v.at[pl.ds((blk + 1) * ncb * 32, ncb * 32)]],
                    rows[b1], sems[b1])
            cps[blk % 2].wait()
            rv = rows[blk % 2]

            def c_body(c, carry1):
                def q_body(q, carry2):
                    sl = pl.ds(q * 16, 16)
                    acc = rv[c * 32, sl]
                    for r in range(1, 32):
                        acc = jnp.maximum(acc, rv[c * 32 + r, sl])
                    out_v[c, sl] = acc
                    return carry2

                lax.fori_loop(0, C // 16, q_body, 0)
                return carry1

            lax.fori_loop(0, ncb, c_body, 0)
            pltpu.sync_copy(
                out_v, out_hbm.at[pl.ds((wid * nblk + blk) * ncb, ncb)])

    return k


def _res_body(m_ref, qc_ref, w1_ref, w2_ref, out_ref):
    fnew = jax.nn.relu(m_ref[0] - qc_ref[0])
    t = jax.nn.relu(jnp.dot(fnew, w1_ref[:], preferred_element_type=jnp.float32))
    out_ref[0] = jax.nn.relu(
        fnew + jnp.dot(t, w2_ref[:], preferred_element_type=jnp.float32))


def _residual(m, qc, W1, W2):
    _, Mc, cout = m.shape
    return pl.pallas_call(
        _res_body,
        grid=(B,),
        in_specs=[pl.BlockSpec((1, Mc, cout), lambda b: (b, 0, 0)),
                  pl.BlockSpec((1, Mc, cout), lambda b: (b, 0, 0)),
                  pl.BlockSpec((cout, 4 * cout), lambda b: (0, 0)),
                  pl.BlockSpec((4 * cout, cout), lambda b: (0, 0))],
        out_specs=pl.BlockSpec((1, Mc, cout), lambda b: (b, 0, 0)),
        out_shape=jax.ShapeDtypeStruct((B, Mc, cout), jnp.float32),
    )(m, qc, W1, W2)


def _dec_body(pf_ref, pcoT_ref, ffc_ref, fsk_ref, wd_ref, out_ref, *, Nc, Nft, Cc):
    pf = pf_ref[0]
    pcoT = pcoT_ref[0]
    d0 = pf[:, 0:1] - pcoT[0:1, :]
    d1 = pf[:, 1:2] - pcoT[1:2, :]
    d2c = pf[:, 2:3] - pcoT[2:3, :]
    e = d0 * d0 + d1 * d1 + d2c * d2c
    iota = lax.broadcasted_iota(jnp.int32, (Nft, Nc), 1)
    WS = jnp.zeros((Nft, Nc), jnp.float32)
    wsum = jnp.zeros((Nft, 1), jnp.float32)
    for t in range(3):
        m = jnp.min(e, axis=1, keepdims=True)
        am = jnp.min(jnp.where(e == m, iota, Nc), axis=1)
        oh = iota == am[:, None]
        w = 1.0 / (m + 1e-8)
        WS = WS + jnp.where(oh, w, 0.0)
        wsum = wsum + w
        if t < 2:
            e = jnp.where(oh, jnp.inf, e)
    WS = WS / wsum
    g = jnp.dot(ffc_ref[0], wd_ref[:Cc, :], preferred_element_type=jnp.float32)
    out_ref[0] = jax.nn.relu(
        jnp.dot(WS, g, preferred_element_type=jnp.float32)
        + jnp.dot(fsk_ref[0], wd_ref[Cc:, :], preferred_element_type=jnp.float32))


def _decoder(pf, pcoT, ffc, fsk, Wd, Nft):
    _, Nf, _ = pf.shape
    Nc = pcoT.shape[2]
    Cc = ffc.shape[2]
    Cs = fsk.shape[2]
    Cd = Wd.shape[1]
    T = Nf // Nft
    body = functools.partial(_dec_body, Nc=Nc, Nft=Nft, Cc=Cc)
    return pl.pallas_call(
        body,
        grid=(B, T),
        in_specs=[pl.BlockSpec((1, Nft, 3), lambda b, t: (b, t, 0)),
                  pl.BlockSpec((1, 3, Nc), lambda b, t: (b, 0, 0)),
                  pl.BlockSpec((1, Nc, Cc), lambda b, t: (b, 0, 0)),
                  pl.BlockSpec((1, Nft, Cs), lambda b, t: (b, t, 0)),
                  pl.BlockSpec((Cc + Cs, Cd), lambda b, t: (0, 0))],
        out_specs=pl.BlockSpec((1, Nft, Cd), lambda b, t: (b, t, 0)),
        out_shape=jax.ShapeDtypeStruct((B, Nf, Cd), jnp.float32),
    )(pf, pcoT, ffc, fsk, Wd)


def _head_body(f_ref, cf_ref, w12_ref, wc_ref, out_ref):
    f = f_ref[0]
    f12 = jnp.dot(f, w12_ref[:], preferred_element_type=jnp.float32)
    out_ref[0, :N, :] = f12[:, :64]
    out_ref[0, N:2 * N, :] = f12[:, 64:]
    out_ref[0, 2 * N:, :] = jnp.dot(cf_ref[0], wc_ref[:],
                                    preferred_element_type=jnp.float32)


def _heads(f, cf, W12, Wc):
    Nt = 2 * N + 16
    return pl.pallas_call(
        _head_body,
        grid=(B,),
        in_specs=[pl.BlockSpec((1, N, 32), lambda b: (b, 0, 0)),
                  pl.BlockSpec((1, 16, 512), lambda b: (b, 0, 0)),
                  pl.BlockSpec((32, 128), lambda b: (0, 0)),
                  pl.BlockSpec((512, 64), lambda b: (0, 0))],
        out_specs=pl.BlockSpec((1, Nt, 64), lambda b: (b, 0, 0)),
        out_shape=jax.ShapeDtypeStruct((B, Nt, 64), jnp.float32),
    )(f, cf, W12, Wc)


_KNN_TILE = [256, 256, 64, 16]
_SC_NCB = [8, 8, 4, 2]
_SC_NBLK = [16, 4, 2, 1]


def kernel(x, W_stem, Wg0, Wg1, Wg2, Wg3, Wr1_0, Wr1_1, Wr1_2, Wr1_3,
           Wr2_0, Wr2_1, Wr2_2, Wr2_3, Wd0, Wd1, Wd2, Wd3, W_f1, W_f2, W_c):
    Wg = [Wg0, Wg1, Wg2, Wg3]
    Wr1 = [Wr1_0, Wr1_1, Wr1_2, Wr1_3]
    Wr2 = [Wr2_0, Wr2_1, Wr2_2, Wr2_3]
    Wd = [Wd0, Wd1, Wd2, Wd3]

    fin = jnp.concatenate([x, x[:, :, 2:]], axis=-1)
    f = _stem(fin, W_stem)
    p = x
    ps = [x]
    fs = [f]
    for i in range(4):
        M = p.shape[1]
        Mc = M // 4
        cout = ENC[i + 1]
        pc = p[:, ::4]
        pT = jnp.swapaxes(p, 1, 2)
        s, qc = _sqc(p, pc, f, Wg[i])
        idx = _knn_topk(pc, pT, NSAMPLE, _KNN_TILE[i])
        Cg = max(cout, 128)
        s_flat = s.reshape(B * M, cout)
        if Cg != cout:
            s_flat = jnp.pad(s_flat, ((0, 0), (0, Cg - cout)))
        gm = _make_sc_gathermax(B * Mc, Cg, _SC_NCB[i], _SC_NBLK[i])
        m = gm(s_flat, idx.reshape(B * Mc * NSAMPLE))
        m = m[:, :cout].reshape(B, Mc, cout)
        f = _residual(m, qc, Wr1[i], Wr2[i])
        p = pc
        ps.append(p)
        fs.append(f)

    c_feats = fs[-1]
    fp, ff = ps[-1], fs[-1]
    for j in range(4):
        p_s, f_s = ps[3 - j], fs[3 - j]
        Nf = p_s.shape[1]
        Nft = min(Nf, 1024)
        pcoT = jnp.swapaxes(fp, 1, 2)
        ff = _decoder(p_s, pcoT, ff, f_s, Wd[j], Nft)
        fp = p_s

    return _heads(ff, c_feats, jnp.concatenate([W_f1, W_f2], axis=1), W_c)
